# baseline (device time: 16297 ns/iter reference)
import jax
import jax.numpy as jnp
from jax import lax
from jax.experimental import pallas as pl
from jax.experimental.pallas import tpu as pltpu

N_DEV = 32
PLANE = 8
NZ = N_DEV // PLANE


def kernel(x):
    m_per, n = x.shape

    def body(
        x_hbm,
        out_ref,
        x_vmem,
        send1,
        recv1,
        send2,
        recv2,
        copy_sem,
        s_sems1,
        r_sems1,
        s_sems2,
        r_sems2,
    ):
        my = lax.axis_index("i")
        base = (my // PLANE) * PLANE
        mi = my % PLANE
        mz = my // PLANE

        barrier = pltpu.get_barrier_semaphore()

        def sig_plane(j, c):
            pl.semaphore_signal(
                barrier,
                inc=1,
                device_id=(base + (mi ^ j),),
                device_id_type=pl.DeviceIdType.MESH,
            )
            return c

        def sig_z(j, c):
            pl.semaphore_signal(
                barrier,
                inc=1,
                device_id=((mz ^ j) * PLANE + mi,),
                device_id_type=pl.DeviceIdType.MESH,
            )
            return c

        lax.fori_loop(1, PLANE, sig_plane, 0)
        lax.fori_loop(1, NZ, sig_z, 0)

        copy = pltpu.make_async_copy(x_hbm, x_vmem, copy_sem)
        copy.start()
        copy.wait()
        partial = jnp.max(x_vmem[...], axis=0, keepdims=True).astype(jnp.bfloat16)
        send1[...] = partial

        pl.semaphore_wait(barrier, (PLANE - 1) + (NZ - 1))

        def rdma1(j):
            return pltpu.make_async_remote_copy(
                src_ref=send1,
                dst_ref=recv1.at[j - 1],
                send_sem=s_sems1.at[j - 1],
                recv_sem=r_sems1.at[j - 1],
                device_id=(base + (mi ^ j),),
                device_id_type=pl.DeviceIdType.MESH,
            )

        lax.fori_loop(1, PLANE, lambda j, c: (rdma1(j).start(), c)[1], 0)
        lax.fori_loop(1, PLANE, lambda j, c: (rdma1(j).wait(), c)[1], 0)

        plane_max = jnp.maximum(
            jnp.max(recv1[...], axis=(0, 1), keepdims=False), partial[0]
        )
        send2[...] = plane_max[None, :]

        def rdma2(j):
            return pltpu.make_async_remote_copy(
                src_ref=send2,
                dst_ref=recv2.at[j - 1],
                send_sem=s_sems2.at[j - 1],
                recv_sem=r_sems2.at[j - 1],
                device_id=((mz ^ j) * PLANE + mi,),
                device_id_type=pl.DeviceIdType.MESH,
            )

        lax.fori_loop(1, NZ, lambda j, c: (rdma2(j).start(), c)[1], 0)
        lax.fori_loop(1, NZ, lambda j, c: (rdma2(j).wait(), c)[1], 0)

        global_max = jnp.maximum(
            jnp.max(recv2[...], axis=(0, 1), keepdims=False), plane_max
        )
        out_ref[...] = global_max.astype(x_vmem.dtype)[None, :]

    return pl.pallas_call(
        body,
        out_shape=jax.ShapeDtypeStruct((1, n), x.dtype),
        in_specs=[pl.BlockSpec(memory_space=pl.ANY)],
        out_specs=pl.BlockSpec(memory_space=pltpu.VMEM),
        scratch_shapes=[
            pltpu.VMEM((m_per, n), x.dtype),
            pltpu.VMEM((1, n), jnp.bfloat16),
            pltpu.VMEM((PLANE - 1, 1, n), jnp.bfloat16),
            pltpu.VMEM((1, n), jnp.bfloat16),
            pltpu.VMEM((NZ - 1, 1, n), jnp.bfloat16),
            pltpu.SemaphoreType.DMA,
            pltpu.SemaphoreType.DMA((PLANE - 1,)),
            pltpu.SemaphoreType.DMA((PLANE - 1,)),
            pltpu.SemaphoreType.DMA((NZ - 1,)),
            pltpu.SemaphoreType.DMA((NZ - 1,)),
        ],
        compiler_params=pltpu.CompilerParams(collective_id=0),
    )(x)
